# baseline (speedup 1.0000x reference)
import functools

import jax
import jax.numpy as jnp
from jax import lax
from jax.experimental import pallas as pl
from jax.experimental.pallas import tpu as pltpu
from jax.experimental.pallas import tpu_sc as plsc

_N = 10000
_E = 320000
_D = 128
_L = 5

_NC = 2
_NS = 16
_NW = _NC * _NS
_CHUNK = 128
_NCH = _E // _CHUNK
_CPW = _NCH // _NW
_EXTRA = _NCH - _CPW * _NW
_NITER = _CPW + 1
_RPT = _N // _NS


def _segsum_body(x_hbm, src_hbm, dst_hbm, zeros_hbm, out_hbm,
                 idx_s, idx_d, rows0, agg_sh, sem0):
    c = lax.axis_index("c")
    s = lax.axis_index("s")
    wid = s * _NC + c

    r0 = s * _RPT
    pltpu.sync_copy(zeros_hbm.at[pl.ds(r0, _RPT)], agg_sh.at[pl.ds(r0, _RPT)])

    base = wid * _CPW
    pltpu.sync_copy(src_hbm.at[pl.ds(base, _CPW)], idx_s.at[pl.ds(0, _CPW)])
    pltpu.sync_copy(dst_hbm.at[pl.ds(base, _CPW)], idx_d.at[pl.ds(0, _CPW)])

    @pl.when(wid < _EXTRA)
    def _():
        xc = _NW * _CPW + wid
        pltpu.sync_copy(src_hbm.at[pl.ds(xc, 1)], idx_s.at[pl.ds(_CPW, 1)])
        pltpu.sync_copy(dst_hbm.at[pl.ds(xc, 1)], idx_d.at[pl.ds(_CPW, 1)])

    nmy = jnp.where(wid < _EXTRA, _CPW + 1, _CPW)

    plsc.subcore_barrier()

    def body(k, carry):
        pltpu.async_copy(x_hbm.at[idx_s.at[k]], rows0, sem0).wait()
        pltpu.sync_copy(rows0, agg_sh.at[idx_d.at[k]], add=True)
        return carry

    lax.fori_loop(0, nmy, body, 0)

    plsc.subcore_barrier()

    pltpu.sync_copy(agg_sh.at[pl.ds(r0, _RPT)], out_hbm.at[c, pl.ds(r0, _RPT)])


_segsum = functools.partial(
    pl.kernel,
    out_type=jax.ShapeDtypeStruct((_NC, _N, _D), jnp.float32),
    mesh=plsc.VectorSubcoreMesh(core_axis_name="c", subcore_axis_name="s",
                                num_cores=_NC, num_subcores=_NS),
    compiler_params=pltpu.CompilerParams(use_tc_tiling_on_sc=False),
    scratch_types=[
        pltpu.VMEM((_NITER, _CHUNK), jnp.int32),
        pltpu.VMEM((_NITER, _CHUNK), jnp.int32),
        pltpu.VMEM((_CHUNK, _D), jnp.float32),
        pltpu.VMEM_SHARED((_N, _D), jnp.float32),
        pltpu.SemaphoreType.DMA,
    ],
)(_segsum_body)


def _bn_relu(t, g, b):
    m = jnp.mean(t, axis=0, keepdims=True)
    v = jnp.mean((t - m) ** 2, axis=0, keepdims=True)
    return jnp.maximum((t - m) / jnp.sqrt(v + 1e-5) * g + b, 0.0)


def _mlp_body(x_ref, p_ref, w1_ref, b1_ref, g1_ref, be1_ref,
              w2_ref, b2_ref, g2_ref, be2_ref, g3_ref, be3_ref, o_ref):
    rst = x_ref[...] + p_ref[0] + p_ref[1]
    t = jnp.dot(rst, w1_ref[...], preferred_element_type=jnp.float32)
    t = _bn_relu(t + b1_ref[...], g1_ref[...], be1_ref[...])
    t = jnp.dot(t, w2_ref[...], preferred_element_type=jnp.float32)
    t = _bn_relu(t + b2_ref[...], g2_ref[...], be2_ref[...])
    o_ref[...] = _bn_relu(t, g3_ref[...], be3_ref[...])


_mlp = pl.pallas_call(
    _mlp_body,
    out_shape=jax.ShapeDtypeStruct((_N, _D), jnp.float32),
    compiler_params=pltpu.CompilerParams(vmem_limit_bytes=100 * 1024 * 1024),
)


def kernel(h, edge_index, W1, b1, g1, be1, W2, b2, g2, be2, g3, be3):
    src2d = edge_index[0].reshape(_NCH, _CHUNK)
    dst2d = edge_index[1].reshape(_NCH, _CHUNK)
    zeros_nd = jnp.zeros((_N, _D), jnp.float32)
    if True:
        MAXDEG = 128
        src = edge_index[0]
        dst = edge_index[1]
        order = jnp.argsort(dst, stable=True)
        sdst = dst[order]
        ssrc = src[order]
        seg_start = jnp.searchsorted(sdst, jnp.arange(_N, dtype=jnp.int32))
        pos = jnp.arange(_E, dtype=jnp.int32) - seg_start[sdst]
        deg = jnp.zeros((_N,), jnp.int32).at[dst].add(1)
        slot = jnp.full((_N, MAXDEG), -1, jnp.int32).at[sdst, pos].set(ssrc)
        x = h
        for i in range(_L):
            def _bn(t, gg, bb):
                m = jnp.mean(t, axis=0)
                v = jnp.mean((t - m) ** 2, axis=0)
                return (t - m) / jnp.sqrt(v + 1e-5) * gg + bb
            agg = jnp.zeros((_N, _D), jnp.float32)
            for k in range(MAXDEG):
                sk = slot[:, k]
                contrib = x[jnp.maximum(sk, 0)]
                agg = jnp.where((sk >= 0)[:, None], agg + contrib, agg)
            rst = x + agg
            t = jax.nn.relu(_bn(rst @ W1[i] + b1[i], g1[i], be1[i]))
            t = jax.nn.relu(_bn(t @ W2[i] + b2[i], g2[i], be2[i]))
            x = jax.nn.relu(_bn(t, g3[i], be3[i]))
        return x
    x = h
    for i in range(_L):
        parts = _segsum(x, src2d, dst2d, zeros_nd)
        if True:
            def _bn(t, gg, bb):
                m = jnp.mean(t, axis=0)
                v = jnp.mean((t - m) ** 2, axis=0)
                return (t - m) / jnp.sqrt(v + 1e-5) * gg + bb
            rst = x + parts[0] + parts[1]
            t = jax.nn.relu(_bn(rst @ W1[i] + b1[i], g1[i], be1[i]))
            t = jax.nn.relu(_bn(t @ W2[i] + b2[i], g2[i], be2[i]))
            x = jax.nn.relu(_bn(t, g3[i], be3[i]))
            continue
        x = _mlp(x, parts,
                 W1[i], b1[i].reshape(1, _D), g1[i].reshape(1, _D),
                 be1[i].reshape(1, _D),
                 W2[i], b2[i].reshape(1, _D), g2[i].reshape(1, _D),
                 be2[i].reshape(1, _D),
                 g3[i].reshape(1, _D), be3[i].reshape(1, _D))
    return x


# speedup vs baseline: 13.2343x; 13.2343x over previous
import functools

import jax
import jax.numpy as jnp
from jax import lax
from jax.experimental import pallas as pl
from jax.experimental.pallas import tpu as pltpu
from jax.experimental.pallas import tpu_sc as plsc

_N = 10000
_E = 320000
_D = 128
_L = 5

_NC = 2
_NS = 16
_CHUNK = 80
_ROWS2D = _E // _CHUNK
_ROWS2D_PAD = 4032
_MAXCH = 126
_RPT = _N // _NS
_NA = _N + 32


def _tile_chunk_base(s):
    return jnp.where(s < 11, 126 * s, 1386 + 123 * (s - 11))


def _tile_chunk_count(s):
    return jnp.where(s < 11, 126, jnp.where(s < 15, 123, 122))


def _segsum_body(x_hbm, src_hbm, dst_hbm, zeros_hbm, out_hbm,
                 idx_s, idx_d, rows0, hd0, hd1, bufp, bufq, agg_sh, sem0):
    c = lax.axis_index("c")
    s = lax.axis_index("s")

    r0 = s * _RPT
    pltpu.sync_copy(zeros_hbm.at[pl.ds(r0, _RPT)], agg_sh.at[pl.ds(r0, _RPT)])
    pltpu.sync_copy(zeros_hbm.at[pl.ds(_N + s, 1)], agg_sh.at[pl.ds(_N + s, 1)])

    crow = 2000 * c + _tile_chunk_base(s)
    nch = _tile_chunk_count(s)
    e0 = crow * _CHUNK

    pltpu.sync_copy(src_hbm.at[pl.ds(crow, _MAXCH)], idx_s)
    pltpu.sync_copy(dst_hbm.at[pl.ds(crow, _MAXCH)], idx_d)

    pltpu.sync_copy(dst_hbm.at[pl.ds(crow, 1)], hd0)
    prev_row = jnp.maximum(crow - 1, 0)
    pltpu.sync_copy(dst_hbm.at[pl.ds(prev_row, 1)], hd1)
    n0 = hd0[0, pl.ds(0, 16)][0]
    prev = hd1[0, pl.ds(_CHUNK - 16, 16)][15]
    is_first = e0 == 0
    head = jnp.logical_and(jnp.logical_not(is_first), prev == n0)
    n0_eff = jnp.where(head, n0, -1)

    spare = jnp.full((16,), _N, jnp.int32) + s

    def rewrite(j, carry):
        for t in range(_CHUNK // 16):
            v = idx_d[j, pl.ds(t * 16, 16)]
            idx_d[j, pl.ds(t * 16, 16)] = jnp.where(v == n0_eff, spare, v)
        return carry

    lax.fori_loop(0, _MAXCH, rewrite, 0)

    plsc.subcore_barrier()

    def body(j, carry):
        pltpu.async_copy(x_hbm.at[idx_s.at[j]], rows0, sem0).wait()
        pltpu.sync_copy(rows0, agg_sh.at[idx_d.at[j]], add=True)
        return carry

    for step in range(_NS):
        @pl.when(s == step)
        def _():
            lax.fori_loop(0, nch, body, 0)
        plsc.subcore_barrier()

    for step in range(_NS):
        @pl.when(jnp.logical_and(s == step, head))
        def _():
            pltpu.sync_copy(agg_sh.at[pl.ds(_N + s, 1)], bufp)
            pltpu.sync_copy(agg_sh.at[pl.ds(n0, 1)], bufq)
            for t in range(_D // 16):
                bufq[0, pl.ds(t * 16, 16)] = (bufq[0, pl.ds(t * 16, 16)]
                                              + bufp[0, pl.ds(t * 16, 16)])
            pltpu.sync_copy(bufq, agg_sh.at[pl.ds(n0, 1)])
        plsc.subcore_barrier()

    pltpu.sync_copy(agg_sh.at[pl.ds(r0, _RPT)], out_hbm.at[c, pl.ds(r0, _RPT)])


_segsum = functools.partial(
    pl.kernel,
    out_type=jax.ShapeDtypeStruct((_NC, _N, _D), jnp.float32),
    mesh=plsc.VectorSubcoreMesh(core_axis_name="c", subcore_axis_name="s",
                                num_cores=_NC, num_subcores=_NS),
    compiler_params=pltpu.CompilerParams(use_tc_tiling_on_sc=False),
    scratch_types=[
        pltpu.VMEM((_MAXCH, _CHUNK), jnp.int32),
        pltpu.VMEM((_MAXCH, _CHUNK), jnp.int32),
        pltpu.VMEM((_CHUNK, _D), jnp.float32),
        pltpu.VMEM((1, _CHUNK), jnp.int32),
        pltpu.VMEM((1, _CHUNK), jnp.int32),
        pltpu.VMEM((1, _D), jnp.float32),
        pltpu.VMEM((1, _D), jnp.float32),
        pltpu.VMEM_SHARED((_NA, _D), jnp.float32),
        pltpu.SemaphoreType.DMA,
    ],
)(_segsum_body)


def _mm_body(a_ref, w_ref, o_ref):
    o_ref[...] = jnp.dot(a_ref[...], w_ref[...],
                         preferred_element_type=jnp.float32)


_mm = pl.pallas_call(
    _mm_body,
    out_shape=jax.ShapeDtypeStruct((_N, _D), jnp.float32),
    compiler_params=pltpu.CompilerParams(vmem_limit_bytes=100 * 1024 * 1024),
)


def kernel(h, edge_index, W1, b1, g1, be1, W2, b2, g2, be2, g3, be3):
    src = edge_index[0]
    dst = edge_index[1]
    order = jnp.argsort(dst, stable=True)
    pad = _ROWS2D_PAD * _CHUNK - _E
    ssrc = jnp.pad(src[order], (0, pad)).reshape(_ROWS2D_PAD, _CHUNK)
    sdst = jnp.pad(dst[order], (0, pad)).reshape(_ROWS2D_PAD, _CHUNK)
    zeros_nd = jnp.zeros((_NA, _D), jnp.float32)
    x = h
    for i in range(_L):
        parts = _segsum(x, ssrc, sdst, zeros_nd)

        def _bn(t, gg, bb):
            m = jnp.mean(t, axis=0)
            v = jnp.mean((t - m) ** 2, axis=0)
            return (t - m) / jnp.sqrt(v + 1e-5) * gg + bb

        agg = parts[0] + parts[1]
        rst = x + agg
        t = jax.nn.relu(_bn(rst @ W1[i] + b1[i], g1[i], be1[i]))
        t = jax.nn.relu(_bn(t @ W2[i] + b2[i], g2[i], be2[i]))
        x = jax.nn.relu(_bn(t, g3[i], be3[i]))
    return x


# speedup vs baseline: 30.9909x; 2.3417x over previous
import functools

import jax
import jax.numpy as jnp
from jax import lax
from jax.experimental import pallas as pl
from jax.experimental.pallas import tpu as pltpu
from jax.experimental.pallas import tpu_sc as plsc

_N = 10000
_E = 320000
_D = 128
_L = 5

_NC = 2
_NS = 16
_CHUNK = 80
_ROWS2D_PAD = 4032
_MAXCH = 126
_RPT = _N // _NS
_NA = _N + 32

_BOUNDS = []
for _c in range(2):
    _off = 160000 * _c
    for _s in range(16):
        _BOUNDS.append(_off)
        _off += 10080 if _s < 11 else (9840 if _s < 15 else 9760)
_BOUNDS.append(320000)


def _tile_chunk_base(s):
    return jnp.where(s < 11, 126 * s, 1386 + 123 * (s - 11))


def _tile_chunk_count(s):
    return jnp.where(s < 11, 126, jnp.where(s < 15, 123, 122))


def _segsum_body(x_hbm, src_hbm, enc_hbm, head_hbm, zeros_hbm, out_hbm,
                 idx_s, enc_v, rows0, rows1, bufp, bufq, agg_sh,
                 sem0, sem1):
    c = lax.axis_index("c")
    s = lax.axis_index("s")

    r0 = s * _RPT
    pltpu.sync_copy(zeros_hbm.at[pl.ds(r0, _RPT)], agg_sh.at[pl.ds(r0, _RPT)])
    pltpu.sync_copy(zeros_hbm.at[pl.ds(_N + s, 1)], agg_sh.at[pl.ds(_N + s, 1)])

    crow = 2000 * c + _tile_chunk_base(s)
    nch = _tile_chunk_count(s)

    pltpu.sync_copy(src_hbm.at[pl.ds(crow, _MAXCH)], idx_s)
    pltpu.sync_copy(enc_hbm.at[pl.ds(crow, _MAXCH)], enc_v)

    plsc.subcore_barrier()

    zero16 = jnp.zeros((16,), jnp.float32)
    nt = _D // 16

    def make_group_body(rows_ref, jref):
        def group_body(g, acc):
            encv = enc_v[jref[0], pl.ds(g * 16, 16)]
            for kk in range(16):
                enc = encv[kk]
                row = g * 16 + kk
                accn = tuple(acc[t] + rows_ref[row, pl.ds(16 * t, 16)]
                             for t in range(nt))

                @pl.when(enc >= 0)
                def _():
                    for t in range(nt):
                        bufq[0, pl.ds(16 * t, 16)] = accn[t]
                    pltpu.sync_copy(bufq, agg_sh.at[pl.ds(enc, 1)])

                acc = tuple(jnp.where(enc >= 0, zero16, a) for a in accn)
            return acc
        return group_body

    pltpu.async_copy(x_hbm.at[idx_s.at[0]], rows0, sem0)

    def chunk_pair(i, acc):
        j = i * 2
        pltpu.async_copy(x_hbm.at[idx_s.at[j + 1]], rows1, sem1)
        pltpu.make_async_copy(x_hbm.at[idx_s.at[j]], rows0, sem0).wait()
        acc = lax.fori_loop(0, jnp.where(j < nch, _CHUNK // 16, 0),
                            make_group_body(rows0, (j,)), acc)

        @pl.when(i + 1 < _MAXCH // 2)
        def _():
            pltpu.async_copy(x_hbm.at[idx_s.at[j + 2]], rows0, sem0)
        pltpu.make_async_copy(x_hbm.at[idx_s.at[j + 1]], rows1, sem1).wait()
        acc = lax.fori_loop(0, jnp.where(j + 1 < nch, _CHUNK // 16, 0),
                            make_group_body(rows1, (j + 1,)), acc)
        return acc

    acc0 = tuple(zero16 for _ in range(nt))
    lax.fori_loop(0, _MAXCH // 2, chunk_pair, acc0)

    plsc.subcore_barrier()

    pltpu.sync_copy(head_hbm.at[pl.ds(16 * c + s, 1)], enc_v.at[pl.ds(0, 1)])
    hvec = enc_v[0, pl.ds(0, 16)]
    for step in range(_NS):
        @pl.when(s == step)
        def _():
            hflag = hvec[0]

            @pl.when(hflag > 0)
            def _():
                n0 = hvec[1]
                pltpu.sync_copy(agg_sh.at[pl.ds(_N + s, 1)], bufp)
                pltpu.sync_copy(agg_sh.at[pl.ds(n0, 1)], bufq)
                for t in range(nt):
                    bufq[0, pl.ds(16 * t, 16)] = (bufq[0, pl.ds(16 * t, 16)]
                                                  + bufp[0, pl.ds(16 * t, 16)])
                pltpu.sync_copy(bufq, agg_sh.at[pl.ds(n0, 1)])
        plsc.subcore_barrier()

    pltpu.sync_copy(agg_sh.at[pl.ds(r0, _RPT)], out_hbm.at[c, pl.ds(r0, _RPT)])


_segsum = functools.partial(
    pl.kernel,
    out_type=jax.ShapeDtypeStruct((_NC, _N, _D), jnp.float32),
    mesh=plsc.VectorSubcoreMesh(core_axis_name="c", subcore_axis_name="s",
                                num_cores=_NC, num_subcores=_NS),
    compiler_params=pltpu.CompilerParams(use_tc_tiling_on_sc=False),
    scratch_types=[
        pltpu.VMEM((_MAXCH, _CHUNK), jnp.int32),
        pltpu.VMEM((_MAXCH, _CHUNK), jnp.int32),
        pltpu.VMEM((_CHUNK, _D), jnp.float32),
        pltpu.VMEM((_CHUNK, _D), jnp.float32),
        pltpu.VMEM((1, _D), jnp.float32),
        pltpu.VMEM((1, _D), jnp.float32),
        pltpu.VMEM_SHARED((_NA, _D), jnp.float32),
        pltpu.SemaphoreType.DMA,
        pltpu.SemaphoreType.DMA,
    ],
)(_segsum_body)


def _preprocess(edge_index):
    src = edge_index[0]
    dst = edge_index[1]
    order = jnp.argsort(dst, stable=True)
    ssrc = src[order]
    sd = dst[order]
    pos = jnp.arange(_E, dtype=jnp.int32)
    barr = jnp.asarray(_BOUNDS, dtype=jnp.int32)
    seg_start = jnp.searchsorted(sd, sd, side="left").astype(jnp.int32)
    w_e = (jnp.searchsorted(barr, pos, side="right") - 1).astype(jnp.int32)
    bw_e = barr[w_e]
    head_e = seg_start < bw_e
    erow = jnp.where(head_e, _N + (w_e % _NS), sd)
    nxt_diff = jnp.concatenate([sd[1:] != sd[:-1],
                                jnp.ones((1,), dtype=bool)])
    bound_next = jnp.isin(pos + 1, barr)
    enc = jnp.where(nxt_diff | bound_next, erow, -1)

    pad = _ROWS2D_PAD * _CHUNK - _E
    ssrc2d = jnp.pad(ssrc, (0, pad)).reshape(_ROWS2D_PAD, _CHUNK)
    enc2d = jnp.pad(enc, (0, pad), constant_values=-1).reshape(
        _ROWS2D_PAD, _CHUNK)

    bw = barr[:_NC * _NS]
    prev = sd[jnp.maximum(bw - 1, 0)]
    first = sd[bw]
    hflag = jnp.where((bw > 0) & (prev == first), 1, -1).astype(jnp.int32)
    head_tbl = jnp.stack([hflag, first], axis=1)
    head_tbl = jnp.pad(head_tbl, ((0, 0), (0, _CHUNK - 2)))
    return ssrc2d, enc2d, head_tbl


def kernel(h, edge_index, W1, b1, g1, be1, W2, b2, g2, be2, g3, be3):
    ssrc2d, enc2d, head_tbl = _preprocess(edge_index)
    zeros_nd = jnp.zeros((_NA, _D), jnp.float32)
    x = h
    for i in range(_L):
        parts = _segsum(x, ssrc2d, enc2d, head_tbl, zeros_nd)

        def _bn(t, gg, bb):
            m = jnp.mean(t, axis=0)
            v = jnp.mean((t - m) ** 2, axis=0)
            return (t - m) / jnp.sqrt(v + 1e-5) * gg + bb

        agg = parts[0] + parts[1]
        rst = x + agg
        t = jax.nn.relu(_bn(rst @ W1[i] + b1[i], g1[i], be1[i]))
        t = jax.nn.relu(_bn(t @ W2[i] + b2[i], g2[i], be2[i]))
        x = jax.nn.relu(_bn(t, g3[i], be3[i]))
    return x


# speedup vs baseline: 122.6525x; 3.9577x over previous
import functools

import jax
import jax.numpy as jnp
from jax import lax
from jax.experimental import pallas as pl
from jax.experimental.pallas import tpu as pltpu
from jax.experimental.pallas import tpu_sc as plsc

_N = 10000
_E = 320000
_D = 128
_L = 5

_NC = 2
_NS = 16
_CHUNK = 80
_ROWS2D_PAD = 4032
_MAXCH = 126
_RPT = _N // _NS
_NA = _N + 32

_BOUNDS = []
for _c in range(2):
    _off = 160000 * _c
    for _s in range(16):
        _BOUNDS.append(_off)
        _off += 10080 if _s < 11 else (9840 if _s < 15 else 9760)
_BOUNDS.append(320000)


def _tile_chunk_base(s):
    return jnp.where(s < 11, 126 * s, 1386 + 123 * (s - 11))


def _tile_chunk_count(s):
    return jnp.where(s < 11, 126, jnp.where(s < 15, 123, 122))


def _segsum_body(x_hbm, src_hbm, enc_hbm, head_hbm, zeros_hbm, out_hbm,
                 idx_s, enc_v, rows0, rows1, bufp, bufq, agg_sh,
                 sem0, sem1):
    c = lax.axis_index("c")
    s = lax.axis_index("s")

    r0 = s * _RPT
    pltpu.sync_copy(zeros_hbm.at[pl.ds(r0, _RPT)], agg_sh.at[pl.ds(r0, _RPT)])
    pltpu.sync_copy(zeros_hbm.at[pl.ds(_N + s, 1)], agg_sh.at[pl.ds(_N + s, 1)])

    crow = 2000 * c + _tile_chunk_base(s)
    nch = _tile_chunk_count(s)

    pltpu.sync_copy(src_hbm.at[pl.ds(crow, _MAXCH)], idx_s)
    pltpu.sync_copy(enc_hbm.at[pl.ds(crow, _MAXCH)], enc_v)

    plsc.subcore_barrier()

    zero16 = jnp.zeros((16,), jnp.float32)
    nt = _D // 16

    def make_group_body(rows_ref, jref):
        def group_body(g, acc):
            encv = enc_v[jref[0], pl.ds(g * 16, 16)]
            for kk in range(16):
                enc = encv[kk]
                row = g * 16 + kk
                accn = tuple(acc[t] + rows_ref[row, pl.ds(16 * t, 16)]
                             for t in range(nt))

                @pl.when(enc >= 0)
                def _():
                    for t in range(nt):
                        bufq[0, pl.ds(16 * t, 16)] = accn[t]
                    pltpu.sync_copy(bufq, agg_sh.at[pl.ds(enc, 1)])

                acc = tuple(jnp.where(enc >= 0, zero16, a) for a in accn)
            return acc
        return group_body

    pltpu.async_copy(x_hbm.at[idx_s.at[0]], rows0, sem0)

    def chunk_pair(i, acc):
        j = i * 2
        pltpu.async_copy(x_hbm.at[idx_s.at[j + 1]], rows1, sem1)
        pltpu.make_async_copy(x_hbm.at[idx_s.at[j]], rows0, sem0).wait()
        acc = lax.fori_loop(0, jnp.where(j < nch, _CHUNK // 16, 0),
                            make_group_body(rows0, (j,)), acc)

        @pl.when(i + 1 < _MAXCH // 2)
        def _():
            pltpu.async_copy(x_hbm.at[idx_s.at[j + 2]], rows0, sem0)
        pltpu.make_async_copy(x_hbm.at[idx_s.at[j + 1]], rows1, sem1).wait()
        acc = lax.fori_loop(0, jnp.where(j + 1 < nch, _CHUNK // 16, 0),
                            make_group_body(rows1, (j + 1,)), acc)
        return acc

    acc0 = tuple(zero16 for _ in range(nt))
    lax.fori_loop(0, _MAXCH // 2, chunk_pair, acc0)

    plsc.subcore_barrier()

    pltpu.sync_copy(head_hbm.at[pl.ds(16 * c + s, 1)], enc_v.at[pl.ds(0, 1)])
    hvec = enc_v[0, pl.ds(0, 16)]
    for step in range(_NS):
        @pl.when(s == step)
        def _():
            hflag = hvec[0]

            @pl.when(hflag > 0)
            def _():
                n0 = hvec[1]
                pltpu.sync_copy(agg_sh.at[pl.ds(_N + s, 1)], bufp)
                pltpu.sync_copy(agg_sh.at[pl.ds(n0, 1)], bufq)
                for t in range(nt):
                    bufq[0, pl.ds(16 * t, 16)] = (bufq[0, pl.ds(16 * t, 16)]
                                                  + bufp[0, pl.ds(16 * t, 16)])
                pltpu.sync_copy(bufq, agg_sh.at[pl.ds(n0, 1)])
        plsc.subcore_barrier()

    pltpu.sync_copy(agg_sh.at[pl.ds(r0, _RPT)], out_hbm.at[c, pl.ds(r0, _RPT)])


_segsum = functools.partial(
    pl.kernel,
    out_type=jax.ShapeDtypeStruct((_NC, _N, _D), jnp.float32),
    mesh=plsc.VectorSubcoreMesh(core_axis_name="c", subcore_axis_name="s",
                                num_cores=_NC, num_subcores=_NS),
    compiler_params=pltpu.CompilerParams(use_tc_tiling_on_sc=False),
    scratch_types=[
        pltpu.VMEM((_MAXCH, _CHUNK), jnp.int32),
        pltpu.VMEM((_MAXCH, _CHUNK), jnp.int32),
        pltpu.VMEM((_CHUNK, _D), jnp.float32),
        pltpu.VMEM((_CHUNK, _D), jnp.float32),
        pltpu.VMEM((1, _D), jnp.float32),
        pltpu.VMEM((1, _D), jnp.float32),
        pltpu.VMEM_SHARED((_NA, _D), jnp.float32),
        pltpu.SemaphoreType.DMA,
        pltpu.SemaphoreType.DMA,
    ],
)(_segsum_body)


def _preprocess(edge_index):
    import numpy as _np
    src = edge_index[0]
    dst = edge_index[1]
    sd, ssrc = lax.sort((dst, src), dimension=0, is_stable=True, num_keys=1)
    pos = jnp.arange(_E, dtype=jnp.int32)
    diff_prev = jnp.concatenate([jnp.ones((1,), dtype=bool),
                                 sd[1:] != sd[:-1]])
    run_start = lax.cummax(jnp.where(diff_prev, pos, -1), axis=0)
    bw_e = jnp.zeros((_E,), jnp.int32)
    s_of = jnp.zeros((_E,), jnp.int32)
    for w in range(_NC * _NS):
        b = _BOUNDS[w]
        bw_e = jnp.where(pos >= b, b, bw_e)
        s_of = jnp.where(pos >= b, w % _NS, s_of)
    head_e = run_start < bw_e
    erow = jnp.where(head_e, _N + s_of, sd)
    nxt_diff = jnp.concatenate([sd[1:] != sd[:-1],
                                jnp.ones((1,), dtype=bool)])
    _bn = _np.zeros((_E,), dtype=bool)
    _bn[_np.asarray(_BOUNDS[1:], dtype=_np.int64) - 1] = True
    bound_next = jnp.asarray(_bn)
    enc = jnp.where(nxt_diff | bound_next, erow, -1)

    pad = _ROWS2D_PAD * _CHUNK - _E
    ssrc2d = jnp.pad(ssrc, (0, pad)).reshape(_ROWS2D_PAD, _CHUNK)
    enc2d = jnp.pad(enc, (0, pad), constant_values=-1).reshape(
        _ROWS2D_PAD, _CHUNK)

    barr = jnp.asarray(_BOUNDS, dtype=jnp.int32)
    bw = barr[:_NC * _NS]
    prev = sd[jnp.maximum(bw - 1, 0)]
    first = sd[bw]
    hflag = jnp.where((bw > 0) & (prev == first), 1, -1).astype(jnp.int32)
    head_tbl = jnp.stack([hflag, first], axis=1)
    head_tbl = jnp.pad(head_tbl, ((0, 0), (0, _CHUNK - 2)))
    return ssrc2d, enc2d, head_tbl


def kernel(h, edge_index, W1, b1, g1, be1, W2, b2, g2, be2, g3, be3):
    ssrc2d, enc2d, head_tbl = _preprocess(edge_index)
    zeros_nd = jnp.zeros((_NA, _D), jnp.float32)
    x = h
    for i in range(_L):
        parts = _segsum(x, ssrc2d, enc2d, head_tbl, zeros_nd)

        def _bn(t, gg, bb):
            m = jnp.mean(t, axis=0)
            v = jnp.mean((t - m) ** 2, axis=0)
            return (t - m) / jnp.sqrt(v + 1e-5) * gg + bb

        agg = parts[0] + parts[1]
        rst = x + agg
        t = jax.nn.relu(_bn(rst @ W1[i] + b1[i], g1[i], be1[i]))
        t = jax.nn.relu(_bn(t @ W2[i] + b2[i], g2[i], be2[i]))
        x = jax.nn.relu(_bn(t, g3[i], be3[i]))
    return x
